# Initial kernel scaffold; baseline (speedup 1.0000x reference)
import jax
import jax.numpy as jnp
from jax.experimental import pallas as pl


def kernel(modified_adj):
    raise NotImplementedError("write your pallas kernel here")


# speedup vs baseline: 3.4502x; 3.4502x over previous
import jax
import jax.numpy as jnp
from jax.experimental import pallas as pl
from jax.experimental.pallas import tpu as pltpu

_BLK = 512


def _degrees_kernel(a_ref, d_ref):
    col_sums = jnp.sum(a_ref[...], axis=0, keepdims=True)
    d_ref[...] = (col_sums == 1.0).astype(jnp.float32)


def _mask_kernel(flags_ref, aij_ref, aji_ref, dj_ref, di_ref, out_ref):
    i = pl.program_id(0)
    j = pl.program_id(1)
    any_flag = (flags_ref[i] | flags_ref[j]) > 0

    @pl.when(jnp.logical_not(any_flag))
    def _():
        out_ref[...] = jnp.ones_like(out_ref)

    @pl.when(any_flag)
    def _():
        term_cols = aij_ref[...] * dj_ref[...]
        term_rows = (aji_ref[...] * di_ref[...]).T
        out_ref[...] = 1.0 - term_cols - term_rows


def _aij_index(i, j, flags):
    need = flags[j] > 0
    return (jnp.where(need, i, 0), jnp.where(need, j, 0))


def _aji_index(i, j, flags):
    need = flags[i] > 0
    return (jnp.where(need, j, 0), jnp.where(need, i, 0))


def kernel(modified_adj):
    n = modified_adj.shape[0]
    t = n // _BLK

    d = pl.pallas_call(
        _degrees_kernel,
        grid=(t,),
        in_specs=[pl.BlockSpec((n, _BLK), lambda j: (0, j))],
        out_specs=pl.BlockSpec((1, _BLK), lambda j: (0, j)),
        out_shape=jax.ShapeDtypeStruct((1, n), jnp.float32),
    )(modified_adj)

    flags = (jnp.max(d.reshape(t, _BLK), axis=1) > 0.0).astype(jnp.int32)

    out = pl.pallas_call(
        _mask_kernel,
        grid_spec=pltpu.PrefetchScalarGridSpec(
            num_scalar_prefetch=1,
            grid=(t, t),
            in_specs=[
                pl.BlockSpec((_BLK, _BLK), _aij_index),
                pl.BlockSpec((_BLK, _BLK), _aji_index),
                pl.BlockSpec((1, _BLK), lambda i, j, flags: (0, j)),
                pl.BlockSpec((1, _BLK), lambda i, j, flags: (0, i)),
            ],
            out_specs=pl.BlockSpec((_BLK, _BLK), lambda i, j, flags: (i, j)),
        ),
        out_shape=jax.ShapeDtypeStruct((n, n), jnp.float32),
    )(flags, modified_adj, modified_adj, d, d)
    return out


# speedup vs baseline: 3.5982x; 1.0429x over previous
import jax
import jax.numpy as jnp
from jax.experimental import pallas as pl
from jax.experimental.pallas import tpu as pltpu

_BLK = 512


def _pass1_kernel(a_ref, d_ref, ones_ref):
    col_sums = jnp.sum(a_ref[...], axis=0, keepdims=True)
    d_ref[...] = (col_sums == 1.0).astype(jnp.float32)[None]
    ones_ref[...] = jnp.ones_like(ones_ref)


def _fix_kernel(num_ref, il_ref, jl_ref, d_ref, a_ref, inout_ref, out_ref,
                aij_s, aji_s, res_s, sem_a, sem_b, sem_o):
    del inout_ref

    def body(r, carry):
        i = il_ref[r]
        j = jl_ref[r]
        cp_a = pltpu.make_async_copy(
            a_ref.at[pl.ds(i * _BLK, _BLK), pl.ds(j * _BLK, _BLK)], aij_s, sem_a)
        cp_b = pltpu.make_async_copy(
            a_ref.at[pl.ds(j * _BLK, _BLK), pl.ds(i * _BLK, _BLK)], aji_s, sem_b)
        cp_a.start()
        cp_b.start()
        cp_a.wait()
        cp_b.wait()
        dj = d_ref[j, 0, :]
        di = d_ref[i, 0, :]
        res_s[...] = 1.0 - aij_s[...] * dj[None, :] - (aji_s[...] * di[None, :]).T
        cp_o = pltpu.make_async_copy(
            res_s, out_ref.at[pl.ds(i * _BLK, _BLK), pl.ds(j * _BLK, _BLK)], sem_o)
        cp_o.start()
        cp_o.wait()
        return carry

    jax.lax.fori_loop(0, num_ref[0], body, 0)


def kernel(modified_adj):
    n = modified_adj.shape[0]
    t = n // _BLK

    d2, ones = pl.pallas_call(
        _pass1_kernel,
        grid=(t,),
        in_specs=[pl.BlockSpec((n, _BLK), lambda j: (0, j))],
        out_specs=[
            pl.BlockSpec((1, 1, _BLK), lambda j: (j, 0, 0)),
            pl.BlockSpec((n, _BLK), lambda j: (0, j)),
        ],
        out_shape=[
            jax.ShapeDtypeStruct((t, 1, _BLK), jnp.float32),
            jax.ShapeDtypeStruct((n, n), jnp.float32),
        ],
    )(modified_adj)

    flags = jnp.max(d2[:, 0, :], axis=1) > 0.0
    need = flags[:, None] | flags[None, :]
    num = jnp.sum(need).astype(jnp.int32).reshape(1)
    ii, jj = jnp.nonzero(need, size=t * t, fill_value=0)

    out = pl.pallas_call(
        _fix_kernel,
        grid=(1,),
        in_specs=[
            pl.BlockSpec(memory_space=pltpu.MemorySpace.SMEM),
            pl.BlockSpec(memory_space=pltpu.MemorySpace.SMEM),
            pl.BlockSpec(memory_space=pltpu.MemorySpace.SMEM),
            pl.BlockSpec((t, 1, _BLK), lambda g: (0, 0, 0)),
            pl.BlockSpec(memory_space=pltpu.MemorySpace.HBM),
            pl.BlockSpec(memory_space=pltpu.MemorySpace.HBM),
        ],
        out_specs=pl.BlockSpec(memory_space=pltpu.MemorySpace.HBM),
        out_shape=jax.ShapeDtypeStruct((n, n), jnp.float32),
        input_output_aliases={5: 0},
        scratch_shapes=[
            pltpu.VMEM((_BLK, _BLK), jnp.float32),
            pltpu.VMEM((_BLK, _BLK), jnp.float32),
            pltpu.VMEM((_BLK, _BLK), jnp.float32),
            pltpu.SemaphoreType.DMA,
            pltpu.SemaphoreType.DMA,
            pltpu.SemaphoreType.DMA,
        ],
    )(num, ii.astype(jnp.int32), jj.astype(jnp.int32), d2, modified_adj, ones)
    return out
